# Initial kernel scaffold; baseline (speedup 1.0000x reference)
import functools

import jax
import jax.numpy as jnp
from jax import lax
from jax.experimental import pallas as pl
from jax.experimental.pallas import tpu as pltpu
from jax.experimental.pallas import tpu_sc as plsc

N = 10000
E = 320000
D = 128

NC = 2
NS = 16
NW = NC * NS
CHUNK = 128
CPW = -(-E // (NW * CHUNK))
E_PAD = CPW * NW * CHUNK
ROWS_PER_SUB = 632
R = ROWS_PER_SUB * NS
DEGW = 16
BLK = 400


def _sc_body(x_hbm, src_hbm, dst_hbm, outs, scratch, with_deg):
    if with_deg:
        parts_hbm, degp_hbm = outs
        sidx, didx, rows, ones, acc, dacc, sem = scratch
    else:
        (parts_hbm,) = outs
        sidx, didx, rows, ones, acc, sem = scratch
        dacc = degp_hbm = None

    cid = lax.axis_index("c")
    sid = lax.axis_index("s")
    wid = sid * NC + cid
    row0 = sid * ROWS_PER_SUB

    def zrow(i, c):
        for j in range(D // 16):
            rows[i, pl.ds(j * 16, 16)] = jnp.zeros((16,), jnp.float32)
        return c

    lax.fori_loop(0, CHUNK, zrow, 0)

    def zone(i, c):
        ones[i, :] = jnp.zeros((DEGW,), jnp.float32)
        return c

    lax.fori_loop(0, CHUNK, zone, 0)

    for k in range(4):
        pltpu.sync_copy(rows, acc.at[pl.ds(row0 + k * CHUNK, CHUNK)])
    pltpu.sync_copy(rows.at[pl.ds(0, ROWS_PER_SUB - 4 * CHUNK)],
                    acc.at[pl.ds(row0 + 4 * CHUNK, ROWS_PER_SUB - 4 * CHUNK)])
    if with_deg:
        for k in range(4):
            pltpu.sync_copy(ones, dacc.at[pl.ds(row0 + k * CHUNK, CHUNK)])
        pltpu.sync_copy(ones.at[pl.ds(0, ROWS_PER_SUB - 4 * CHUNK)],
                        dacc.at[pl.ds(row0 + 4 * CHUNK, ROWS_PER_SUB - 4 * CHUNK)])

        def fone(i, c):
            ones[i, :] = jnp.ones((DEGW,), jnp.float32)
            return c

        lax.fori_loop(0, CHUNK, fone, 0)

    plsc.subcore_barrier()

    def body(c, carry):
        base = pl.multiple_of((wid * CPW + c) * CHUNK, CHUNK)
        pltpu.sync_copy(src_hbm.at[pl.ds(base, CHUNK)], sidx)
        pltpu.sync_copy(dst_hbm.at[pl.ds(base, CHUNK)], didx)
        pltpu.async_copy(x_hbm.at[sidx], rows, sem).wait()
        pltpu.sync_copy(rows, acc.at[didx], add=True)
        if with_deg:
            pltpu.sync_copy(ones, dacc.at[didx], add=True)
        return carry

    lax.fori_loop(0, CPW, body, 0)
    plsc.subcore_barrier()

    pltpu.sync_copy(acc.at[pl.ds(row0, ROWS_PER_SUB)],
                    parts_hbm.at[cid, pl.ds(row0, ROWS_PER_SUB)])
    if with_deg:
        pltpu.sync_copy(dacc.at[pl.ds(row0, ROWS_PER_SUB)],
                        degp_hbm.at[cid, pl.ds(row0, ROWS_PER_SUB)])


def _sc_agg_deg(x, srcp, dstp):
    mesh = plsc.VectorSubcoreMesh(core_axis_name="c", subcore_axis_name="s")

    @functools.partial(
        pl.kernel,
        out_type=(jax.ShapeDtypeStruct((NC, R, D), jnp.float32),
                  jax.ShapeDtypeStruct((NC, R, DEGW), jnp.float32)),
        mesh=mesh,
        scratch_types=[
            pltpu.VMEM((CHUNK,), jnp.int32),
            pltpu.VMEM((CHUNK,), jnp.int32),
            pltpu.VMEM((CHUNK, D), jnp.float32),
            pltpu.VMEM((CHUNK, DEGW), jnp.float32),
            pltpu.VMEM_SHARED((R, D), jnp.float32),
            pltpu.VMEM_SHARED((R, DEGW), jnp.float32),
            pltpu.SemaphoreType.DMA,
        ],
    )
    def k(x_hbm, src_hbm, dst_hbm, parts_hbm, degp_hbm, *scratch):
        _sc_body(x_hbm, src_hbm, dst_hbm, (parts_hbm, degp_hbm), scratch, True)

    return k(x, srcp, dstp)


def _sc_agg(h, srcp, dstp):
    mesh = plsc.VectorSubcoreMesh(core_axis_name="c", subcore_axis_name="s")

    @functools.partial(
        pl.kernel,
        out_type=jax.ShapeDtypeStruct((NC, R, D), jnp.float32),
        mesh=mesh,
        scratch_types=[
            pltpu.VMEM((CHUNK,), jnp.int32),
            pltpu.VMEM((CHUNK,), jnp.int32),
            pltpu.VMEM((CHUNK, D), jnp.float32),
            pltpu.VMEM((CHUNK, DEGW), jnp.float32),
            pltpu.VMEM_SHARED((R, D), jnp.float32),
            pltpu.SemaphoreType.DMA,
        ],
    )
    def k(h_hbm, src_hbm, dst_hbm, parts_hbm, *scratch):
        _sc_body(h_hbm, src_hbm, dst_hbm, (parts_hbm,), scratch, False)

    return k(h, srcp, dstp)


def _tc_layer(parts, degp, h, WlT, bl, WrT, last):
    def body(p_ref, d_ref, h_ref, wl_ref, b_ref, wr_ref, o_ref):
        p = p_ref[0] + p_ref[1]
        d = d_ref[0, :, 0:1] + d_ref[1, :, 0:1]
        mean = p / jnp.maximum(d, 1.0)
        o = (jnp.dot(mean, wl_ref[...], preferred_element_type=jnp.float32)
             + b_ref[...]
             + jnp.dot(h_ref[...], wr_ref[...], preferred_element_type=jnp.float32))
        if last:
            m = jnp.max(o, axis=1, keepdims=True)
            e = jnp.exp(o - m)
            s = jnp.sum(e, axis=1, keepdims=True)
            o_ref[...] = (o - m) - jnp.log(s)
        else:
            o_ref[...] = jnp.maximum(o, 0.0)

    return pl.pallas_call(
        body,
        grid=(N // BLK,),
        in_specs=[
            pl.BlockSpec((NC, BLK, D), lambda i: (0, i, 0)),
            pl.BlockSpec((NC, BLK, DEGW), lambda i: (0, i, 0)),
            pl.BlockSpec((BLK, D), lambda i: (i, 0)),
            pl.BlockSpec((D, D), lambda i: (0, 0)),
            pl.BlockSpec((1, D), lambda i: (0, 0)),
            pl.BlockSpec((D, D), lambda i: (0, 0)),
        ],
        out_specs=pl.BlockSpec((BLK, D), lambda i: (i, 0)),
        out_shape=jax.ShapeDtypeStruct((N, D), jnp.float32),
    )(parts, degp, h, WlT, bl, WrT)


def kernel(x, edge_index, W1l, b1l, W1r, W2l, b2l, W2r):
    src = edge_index[0].astype(jnp.int32)
    dst = edge_index[1].astype(jnp.int32)
    pad = E_PAD - E
    srcp = jnp.concatenate([src, jnp.zeros((pad,), jnp.int32)])
    dstp = jnp.concatenate([dst, jnp.full((pad,), N, jnp.int32)])

    parts1, degp = _sc_agg_deg(x, srcp, dstp)
    h = _tc_layer(parts1, degp, x, W1l.T, b1l.reshape(1, D), W1r.T, last=False)
    parts2 = _sc_agg(h, srcp, dstp)
    out = _tc_layer(parts2, degp, h, W2l.T, b2l.reshape(1, D), W2r.T, last=True)
    return out


# speedup vs baseline: 3.6799x; 3.6799x over previous
import functools

import jax
import jax.numpy as jnp
from jax import lax
from jax.experimental import pallas as pl
from jax.experimental.pallas import tpu as pltpu
from jax.experimental.pallas import tpu_sc as plsc

N = 10000
E = 320000
D = 128

NC = 2
NS = 16
NW = NC * NS
L = 16
CHUNK = 128
CPW = -(-E // (NW * CHUNK))
E_PAD = CPW * NW * CHUNK
ROWS_PER_SUB = 640
R = ROWS_PER_SUB * NS
BLK = 400

def _mesh():
    return plsc.VectorSubcoreMesh(core_axis_name="c", subcore_axis_name="s")


def _zero_acc_slice(rows, acc, row0):
    def zrow(i, c):
        for j in range(D // L):
            rows[i, pl.ds(j * L, L)] = jnp.zeros((L,), jnp.float32)
        return c

    lax.fori_loop(0, CHUNK, zrow, 0)
    for blk in range(ROWS_PER_SUB // CHUNK):
        pltpu.sync_copy(rows, acc.at[pl.ds(row0 + blk * CHUNK, CHUNK)])


def _sc_agg(h, srcp, dstp):
    @functools.partial(
        pl.kernel,
        out_type=jax.ShapeDtypeStruct((NC, R, D), jnp.float32),
        mesh=_mesh(),
        scratch_types=[
            pltpu.VMEM((CHUNK,), jnp.int32),
            pltpu.VMEM((CHUNK,), jnp.int32),
            pltpu.VMEM((CHUNK, D), jnp.float32),
            pltpu.VMEM_SHARED((R, D), jnp.float32),
            pltpu.SemaphoreType.DMA,
        ],
    )
    def k(x_hbm, src_hbm, dst_hbm, parts_hbm, sidx, didx, rows, acc, sem):
        cid = lax.axis_index("c")
        sid = lax.axis_index("s")
        wid = sid * NC + cid
        row0 = sid * ROWS_PER_SUB

        _zero_acc_slice(rows, acc, row0)
        plsc.subcore_barrier()

        def body(c, carry):
            base = pl.multiple_of((wid * CPW + c) * CHUNK, CHUNK)
            pltpu.sync_copy(src_hbm.at[pl.ds(base, CHUNK)], sidx)
            pltpu.sync_copy(dst_hbm.at[pl.ds(base, CHUNK)], didx)
            pltpu.async_copy(x_hbm.at[sidx], rows, sem).wait()
            pltpu.sync_copy(rows, acc.at[didx], add=True)
            return carry

        lax.fori_loop(0, CPW, body, 0)
        plsc.subcore_barrier()
        pltpu.sync_copy(acc.at[pl.ds(row0, ROWS_PER_SUB)],
                        parts_hbm.at[cid, pl.ds(row0, ROWS_PER_SUB)])

    return k(h, srcp, dstp)


def _sc_deg(dstp):
    @functools.partial(
        pl.kernel,
        out_type=jax.ShapeDtypeStruct((NC, R, D), jnp.float32),
        mesh=_mesh(),
        scratch_types=[
            pltpu.VMEM((CHUNK,), jnp.int32),
            pltpu.VMEM((CHUNK, D), jnp.float32),
            pltpu.VMEM_SHARED((R, D), jnp.float32),
        ],
    )
    def k(dst_hbm, degp_hbm, didx, rows, acc):
        cid = lax.axis_index("c")
        sid = lax.axis_index("s")
        wid = sid * NC + cid
        row0 = sid * ROWS_PER_SUB

        _zero_acc_slice(rows, acc, row0)

        def orow(i, c):
            for j in range(D // L):
                rows[i, pl.ds(j * L, L)] = jnp.ones((L,), jnp.float32)
            return c

        lax.fori_loop(0, CHUNK, orow, 0)
        plsc.subcore_barrier()

        def body(c, carry):
            base = pl.multiple_of((wid * CPW + c) * CHUNK, CHUNK)
            pltpu.sync_copy(dst_hbm.at[pl.ds(base, CHUNK)], didx)
            pltpu.sync_copy(rows, acc.at[didx], add=True)
            return carry

        lax.fori_loop(0, CPW, body, 0)
        plsc.subcore_barrier()
        pltpu.sync_copy(acc.at[pl.ds(row0, ROWS_PER_SUB)],
                        degp_hbm.at[cid, pl.ds(row0, ROWS_PER_SUB)])

    return k(dstp)


def _tc_layer(parts, degp, h, WlT, bl, WrT, last):
    def body(p_ref, d_ref, h_ref, wl_ref, b_ref, wr_ref, o_ref):
        p = p_ref[0] + p_ref[1]
        d = d_ref[0, :, 0:1] + d_ref[1, :, 0:1]
        mean = p / jnp.maximum(d, 1.0)
        o = (jnp.dot(mean, wl_ref[...], preferred_element_type=jnp.float32)
             + b_ref[...]
             + jnp.dot(h_ref[...], wr_ref[...], preferred_element_type=jnp.float32))
        if last:
            m = jnp.max(o, axis=1, keepdims=True)
            e = jnp.exp(o - m)
            s = jnp.sum(e, axis=1, keepdims=True)
            o_ref[...] = (o - m) - jnp.log(s)
        else:
            o_ref[...] = jnp.maximum(o, 0.0)

    return pl.pallas_call(
        body,
        grid=(N // BLK,),
        in_specs=[
            pl.BlockSpec((NC, BLK, D), lambda i: (0, i, 0)),
            pl.BlockSpec((NC, BLK, D), lambda i: (0, i, 0)),
            pl.BlockSpec((BLK, D), lambda i: (i, 0)),
            pl.BlockSpec((D, D), lambda i: (0, 0)),
            pl.BlockSpec((1, D), lambda i: (0, 0)),
            pl.BlockSpec((D, D), lambda i: (0, 0)),
        ],
        out_specs=pl.BlockSpec((BLK, D), lambda i: (i, 0)),
        out_shape=jax.ShapeDtypeStruct((N, D), jnp.float32),
    )(parts, degp, h, WlT, bl, WrT)


def kernel(x, edge_index, W1l, b1l, W1r, W2l, b2l, W2r):
    src = edge_index[0].astype(jnp.int32)
    dst = edge_index[1].astype(jnp.int32)
    pad = E_PAD - E
    srcp = jnp.concatenate([src, jnp.zeros((pad,), jnp.int32)])
    dstp = jnp.concatenate([dst, jnp.full((pad,), N, jnp.int32)])

    degp = _sc_deg(dstp)
    parts1 = _sc_agg(x, srcp, dstp)
    h = _tc_layer(parts1, degp, x, W1l.T, b1l.reshape(1, D), W1r.T, last=False)
    parts2 = _sc_agg(h, srcp, dstp)
    out = _tc_layer(parts2, degp, h, W2l.T, b2l.reshape(1, D), W2r.T, last=True)
    return out
